# baseline (speedup 1.0000x reference)
import functools

import jax
import jax.numpy as jnp
from jax import lax
from jax.experimental import pallas as pl
from jax.experimental.pallas import tpu as pltpu
from jax.experimental.pallas import tpu_sc as plsc

TYPE_SIZE = 16
EMBED = 128
NW = 32
CHUNK = 80
NBUF = 8
AHEAD = 6


def _make_lookup(total):
    per_w = total // NW
    iters = per_w // CHUNK
    assert iters % NBUF == 0 and CHUNK % 8 == 0
    mesh = plsc.VectorSubcoreMesh(core_axis_name="c", subcore_axis_name="s")

    @functools.partial(
        pl.kernel,
        mesh=mesh,
        out_type=jax.ShapeDtypeStruct((total, EMBED), jnp.float32),
        scratch_types=[
            pltpu.VMEM_SHARED((TYPE_SIZE, EMBED), jnp.float32),
            pltpu.VMEM((per_w,), jnp.int32),
            pltpu.VMEM((NBUF, CHUNK, EMBED), jnp.float32),
            pltpu.SemaphoreType.DMA((NBUF,)),
            pltpu.SemaphoreType.DMA((NBUF,)),
        ],
    )
    def lookup(idx_hbm, table_hbm, out_hbm, table_sh, idx_v, rows_v, gsem, wsem):
        wid = lax.axis_index("s") * 2 + lax.axis_index("c")
        base = pl.multiple_of(wid * per_w, 8)
        pltpu.sync_copy(idx_hbm.at[pl.ds(base, per_w)], idx_v)

        @pl.when(lax.axis_index("s") == 0)
        def _():
            pltpu.sync_copy(table_hbm, table_sh)

        plsc.subcore_barrier()

        def idx_at(it):
            return idx_v.at[pl.ds(pl.multiple_of(it * CHUNK, 8), CHUNK)]

        def gather(it, b):
            return pltpu.make_async_copy(
                table_sh.at[idx_at(it)], rows_v.at[b], gsem.at[b]
            )

        def write(it, b):
            off = pl.multiple_of(base + it * CHUNK, 8)
            return pltpu.make_async_copy(
                rows_v.at[b], out_hbm.at[pl.ds(off, CHUNK)], wsem.at[b]
            )

        for p in range(AHEAD):
            gather(p, p).start()

        def group(g, carry):
            for b in range(NBUF):
                it = g * NBUF + b
                gather(it, b).wait()
                nxt = it + AHEAD
                bf = (b + AHEAD) % NBUF

                @pl.when(nxt < iters)
                def _():
                    @pl.when(nxt >= NBUF)
                    def _():
                        write(nxt - NBUF, bf).wait()

                    gather(nxt, bf).start()

                write(it, b).start()
            return carry

        lax.fori_loop(0, iters // NBUF, group, 0)

        for b in range(NBUF):
            write(iters - NBUF + b, b).wait()

    return lookup


def kernel(type_ids, table):
    b, l = type_ids.shape
    total = b * l
    flat = type_ids.reshape(total)
    out = _make_lookup(total)(flat, table)
    return out.reshape(b, l, EMBED)


# speedup vs baseline: 1.0006x; 1.0006x over previous
import functools

import jax
import jax.numpy as jnp
from jax import lax
from jax.experimental import pallas as pl
from jax.experimental.pallas import tpu as pltpu
from jax.experimental.pallas import tpu_sc as plsc

TYPE_SIZE = 16
EMBED = 128
NW = 32
CHUNK = 80
NBUF = 8
AHEAD = 4


def _make_lookup(total):
    per_w = total // NW
    iters = per_w // CHUNK
    assert iters % NBUF == 0 and CHUNK % 8 == 0
    mesh = plsc.VectorSubcoreMesh(core_axis_name="c", subcore_axis_name="s")

    @functools.partial(
        pl.kernel,
        mesh=mesh,
        out_type=jax.ShapeDtypeStruct((total, EMBED), jnp.float32),
        scratch_types=[
            pltpu.VMEM_SHARED((TYPE_SIZE, EMBED), jnp.float32),
            pltpu.VMEM((per_w,), jnp.int32),
            pltpu.VMEM((NBUF, CHUNK, EMBED), jnp.float32),
            pltpu.SemaphoreType.DMA((NBUF,)),
            pltpu.SemaphoreType.DMA((NBUF,)),
        ],
    )
    def lookup(idx_hbm, table_hbm, out_hbm, table_sh, idx_v, rows_v, gsem, wsem):
        wid = lax.axis_index("s") * 2 + lax.axis_index("c")
        base = pl.multiple_of(wid * per_w, 8)
        pltpu.sync_copy(idx_hbm.at[pl.ds(base, per_w)], idx_v)

        @pl.when(lax.axis_index("s") == 0)
        def _():
            pltpu.sync_copy(table_hbm, table_sh)

        plsc.subcore_barrier()

        def idx_at(it):
            return idx_v.at[pl.ds(pl.multiple_of(it * CHUNK, 8), CHUNK)]

        def gather(it, b):
            return pltpu.make_async_copy(
                table_sh.at[idx_at(it)], rows_v.at[b], gsem.at[b]
            )

        def write(it, b):
            off = pl.multiple_of(base + it * CHUNK, 8)
            return pltpu.make_async_copy(
                rows_v.at[b], out_hbm.at[pl.ds(off, CHUNK)], wsem.at[b]
            )

        for p in range(AHEAD):
            gather(p, p).start()

        def group(g, carry):
            for b in range(NBUF):
                it = g * NBUF + b
                gather(it, b).wait()
                nxt = it + AHEAD
                bf = (b + AHEAD) % NBUF

                @pl.when(nxt < iters)
                def _():
                    @pl.when(nxt >= NBUF)
                    def _():
                        write(nxt - NBUF, bf).wait()

                    gather(nxt, bf).start()

                write(it, b).start()
            return carry

        lax.fori_loop(0, iters // NBUF, group, 0)

        for b in range(NBUF):
            write(iters - NBUF + b, b).wait()

    return lookup


def kernel(type_ids, table):
    b, l = type_ids.shape
    total = b * l
    flat = type_ids.reshape(total)
    out = _make_lookup(total)(flat, table)
    return out.reshape(b, l, EMBED)
